# baseline (speedup 1.0000x reference)
import functools

import jax
import jax.numpy as jnp
from jax import lax
from jax.experimental import pallas as pl
from jax.experimental.pallas import tpu as pltpu
from jax.experimental.pallas import tpu_sc as plsc

B, S, H = 64, 512, 768
V, VD = 196, 2048
MAXP, TV = 512, 2
EPS = 1e-12
NB = 4
CH = 32
SCCHUNKS = ((0, 32), (32, 32), (64, 32), (96, 32), (128, 32), (160, 32),
            (192, 8))
VPAD = 200
SPAD = 712
TXT = 64
LANES = 16


def _ln(x, gamma, beta):
    mean = jnp.mean(x, axis=-1, keepdims=True)
    xc = x - mean
    var = jnp.mean(xc * xc, axis=-1, keepdims=True)
    return xc * lax.rsqrt(var + EPS) * gamma + beta


def _tc_body(tt_ids_ref, vtt_ids_ref, ttt_ref, vttt_ref, w_ref, bias_ref,
             gamma_ref, beta_ref, ie_ref, ve_ref, out_emb_ref):
    gamma = gamma_ref[0]
    beta = beta_ref[0]

    ids = tt_ids_ref[0]
    m1 = (ids == 1).astype(jnp.float32)[:, :, None]
    tte = ttt_ref[0] * (1.0 - m1) + ttt_ref[1] * m1
    out_emb_ref[:, :S, :] = _ln(ie_ref[:, :, :] + tte, gamma, beta)

    vids = vtt_ids_ref[0]
    w = w_ref[:, :].astype(jnp.bfloat16)
    for k in range(NB):
        vm1 = (vids[k] == 1).astype(jnp.float32)[:, None]
        vtt = vttt_ref[0] * (1.0 - vm1) + vttt_ref[1] * vm1
        vis = jnp.dot(ve_ref[k].astype(jnp.bfloat16), w,
                      preferred_element_type=jnp.float32) + bias_ref[0]
        out_emb_ref[k, S:, :] = _ln(vis + vtt, gamma, beta)


def _tail_body(f0_ref, f1_ref, pos_ref, vp_ref, alias_ref, out_ref):
    n = f0_ref.shape[0]
    iota = lax.broadcasted_iota(jnp.int32, (n, MAXP), 1)
    cnt = ((f0_ref[:, :] == iota).astype(jnp.float32) +
           (f1_ref[:, :] == iota).astype(jnp.float32))
    vpe = jnp.dot(cnt, pos_ref[:, :],
                  preferred_element_type=jnp.float32) * 0.5 + vp_ref[0]
    out_ref[:, :, :] = vpe.reshape(B, n // B, H)


def _sc_body(pos_hbm, ita0_hbm, ita1_hbm, vp0_hbm, out_hbm,
             idx0_v, idx1_v, r0_v, r1_v, res_v, txt_v, vp_v,
             sem0, sem1, semt):
    info = plsc.get_sparse_core_info()
    nc = info.num_cores
    wid = lax.axis_index("s") * nc + lax.axis_index("c")
    per_w = B // (nc * info.num_subcores)

    pltpu.sync_copy(vp0_hbm, vp_v)

    def text_body(t, carry):
        toff = pl.multiple_of(t * TXT, TXT)
        pltpu.sync_copy(pos_hbm.at[pl.ds(toff, TXT), :], txt_v)
        for bi in range(per_w):
            b = wid * per_w + bi
            pltpu.sync_copy(txt_v, out_hbm.at[b, pl.ds(toff, TXT), :])
        return carry

    lax.fori_loop(0, S // TXT, text_body, 0)

    for bi in range(per_w):
        b = wid * per_w + bi
        pltpu.sync_copy(ita0_hbm.at[b], idx0_v)
        pltpu.sync_copy(ita1_hbm.at[b], idx1_v)

        def chunk_body(c, carry, b=b):
            off = pl.multiple_of(c * CH, CH)
            g0 = pltpu.async_copy(pos_hbm.at[idx0_v.at[pl.ds(off, CH)]],
                                  r0_v, sem0)
            g1 = pltpu.async_copy(pos_hbm.at[idx1_v.at[pl.ds(off, CH)]],
                                  r1_v, sem1)
            g0.wait()
            g1.wait()

            def row_body(r, rcarry):
                for k in range(H // LANES):
                    sl = pl.ds(k * LANES, LANES)
                    res_v[r, sl] = (r0_v[r, sl] + r1_v[r, sl]) * 0.5 + vp_v[sl]
                return rcarry

            lax.fori_loop(0, CH, row_body, 0)
            pltpu.sync_copy(res_v, out_hbm.at[b, pl.ds(S + off, CH), :])
            return carry

        lax.fori_loop(0, V // CH, chunk_body, 0)


def _make_sc_call():
    mesh = plsc.VectorSubcoreMesh(core_axis_name="c", subcore_axis_name="s")
    return pl.kernel(
        _sc_body,
        mesh=mesh,
        out_type=jax.ShapeDtypeStruct((B, S + V, H), jnp.float32),
        scratch_types=[
            pltpu.VMEM((VPAD,), jnp.int32),
            pltpu.VMEM((VPAD,), jnp.int32),
            pltpu.VMEM((CH, H), jnp.float32),
            pltpu.VMEM((CH, H), jnp.float32),
            pltpu.VMEM((CH, H), jnp.float32),
            pltpu.VMEM((TXT, H), jnp.float32),
            pltpu.VMEM((H,), jnp.float32),
            pltpu.SemaphoreType.DMA,
            pltpu.SemaphoreType.DMA,
            pltpu.SemaphoreType.DMA,
        ],
    )


def kernel(inputs_embeds, token_type_ids, visual_embeds, visual_token_type_ids,
           image_text_alignment, pos_table, tok_type_table, vis_tok_type_table,
           vis_pos_table, vis_proj_W, vis_proj_b, ln_gamma, ln_beta):
    ita = image_text_alignment.astype(jnp.int32)
    pad = jnp.zeros((B, VPAD - V), dtype=jnp.int32)
    ita0 = jnp.concatenate([ita[:, :, 0], pad], axis=1)
    ita1 = jnp.concatenate([ita[:, :, 1], pad], axis=1)

    out_pos = _make_sc_call()(pos_table, ita0, ita1, vis_pos_table[0])

    out_emb = pl.pallas_call(
        _tc_body,
        grid=(B // NB,),
        in_specs=[
            pl.BlockSpec((1, NB, S), lambda g: (g, 0, 0)),
            pl.BlockSpec((1, NB, V), lambda g: (g, 0, 0)),
            pl.BlockSpec((TV, H), lambda g: (0, 0)),
            pl.BlockSpec((TV, H), lambda g: (0, 0)),
            pl.BlockSpec((VD, H), lambda g: (0, 0)),
            pl.BlockSpec((1, H), lambda g: (0, 0)),
            pl.BlockSpec((1, H), lambda g: (0, 0)),
            pl.BlockSpec((1, H), lambda g: (0, 0)),
            pl.BlockSpec((NB, S, H), lambda g: (g, 0, 0)),
            pl.BlockSpec((NB, V, VD), lambda g: (g, 0, 0)),
        ],
        out_specs=pl.BlockSpec((NB, S + V, H), lambda g: (g, 0, 0)),
        out_shape=jax.ShapeDtypeStruct((B, S + V, H), jnp.float32),
    )(
        token_type_ids.astype(jnp.int32).reshape(B // NB, NB, S),
        visual_token_type_ids.astype(jnp.int32).reshape(B // NB, NB, V),
        tok_type_table, vis_tok_type_table, vis_proj_W,
        vis_proj_b.reshape(1, H), ln_gamma.reshape(1, H), ln_beta.reshape(1, H),
        inputs_embeds, visual_embeds,
    )

    tails = SPAD - S - CH * (V // CH)
    f0 = ita0[:, CH * (V // CH):CH * (V // CH) + tails].reshape(B * tails, 1)
    f1 = ita1[:, CH * (V // CH):CH * (V // CH) + tails].reshape(B * tails, 1)
    out_pos = pl.pallas_call(
        _tail_body,
        grid=(1,),
        in_specs=[
            pl.BlockSpec((B * tails, 1), lambda g: (0, 0)),
            pl.BlockSpec((B * tails, 1), lambda g: (0, 0)),
            pl.BlockSpec((MAXP, H), lambda g: (0, 0)),
            pl.BlockSpec((1, H), lambda g: (0, 0)),
            pl.BlockSpec(memory_space=pltpu.MemorySpace.HBM),
        ],
        out_specs=pl.BlockSpec((B, tails, H),
                               lambda g: (0, (S + CH * (V // CH)) // tails, 0)),
        out_shape=jax.ShapeDtypeStruct((B, S + V, H), jnp.float32),
        input_output_aliases={4: 0},
    )(f0, f1, pos_table, vis_pos_table[0:1], out_pos)
    return (out_emb, out_pos)


# speedup vs baseline: 1.0530x; 1.0530x over previous
import functools

import jax
import jax.numpy as jnp
from jax import lax
from jax.experimental import pallas as pl
from jax.experimental.pallas import tpu as pltpu
from jax.experimental.pallas import tpu_sc as plsc

B, S, H = 64, 512, 768
V, VD = 196, 2048
MAXP, TV = 512, 2
EPS = 1e-12
NB = 4
CH = 32
SCCHUNKS = ((0, 32), (32, 32), (64, 32), (96, 32), (128, 32), (160, 32),
            (192, 8))
VPAD = 200
SPAD = 712
TXT = 64
LANES = 16


def _ln(x, gamma, beta):
    mean = jnp.mean(x, axis=-1, keepdims=True)
    xc = x - mean
    var = jnp.mean(xc * xc, axis=-1, keepdims=True)
    return xc * lax.rsqrt(var + EPS) * gamma + beta


def _tc_body(tt_ids_ref, vtt_ids_ref, ttt_ref, vttt_ref, w_ref, bias_ref,
             gamma_ref, beta_ref, ie_ref, ve_ref, out_emb_ref):
    gamma = gamma_ref[0]
    beta = beta_ref[0]

    ids = tt_ids_ref[0]
    m1 = (ids == 1).astype(jnp.float32)[:, :, None]
    tte = ttt_ref[0] * (1.0 - m1) + ttt_ref[1] * m1
    out_emb_ref[:, :S, :] = _ln(ie_ref[:, :, :] + tte, gamma, beta)

    vids = vtt_ids_ref[0]
    w = w_ref[:, :].astype(jnp.bfloat16)
    for k in range(NB):
        vm1 = (vids[k] == 1).astype(jnp.float32)[:, None]
        vtt = vttt_ref[0] * (1.0 - vm1) + vttt_ref[1] * vm1
        vis = jnp.dot(ve_ref[k].astype(jnp.bfloat16), w,
                      preferred_element_type=jnp.float32) + bias_ref[0]
        out_emb_ref[k, S:, :] = _ln(vis + vtt, gamma, beta)


def _tail_body(f0_ref, f1_ref, pos_ref, vp_ref, alias_ref, out_ref):
    n = f0_ref.shape[0]
    iota = lax.broadcasted_iota(jnp.int32, (n, MAXP), 1)
    cnt = ((f0_ref[:, :] == iota).astype(jnp.float32) +
           (f1_ref[:, :] == iota).astype(jnp.float32))
    vpe = jnp.dot(cnt, pos_ref[:, :],
                  preferred_element_type=jnp.float32) * 0.5 + vp_ref[0]
    out_ref[:, :, :] = vpe.reshape(B, n // B, H)


def _sc_body(pos_hbm, ita0_hbm, ita1_hbm, vp0_hbm, out_hbm,
             idx0_v, idx1_v, r0_v, r1_v, res_v, txt_v, vp_v,
             sem0, sem1, semt):
    info = plsc.get_sparse_core_info()
    nc = info.num_cores
    nw = nc * info.num_subcores
    wid = lax.axis_index("s") * nc + lax.axis_index("c")
    half = nw // 2
    per_w = B // half


    @pl.when(wid < half)
    def _text():
        def text_body(t, carry):
            toff = pl.multiple_of(t * TXT, TXT)
            pltpu.sync_copy(pos_hbm.at[pl.ds(toff, TXT), :], txt_v)
            hs = [pltpu.async_copy(txt_v,
                                   out_hbm.at[wid * per_w + bi,
                                              pl.ds(toff, TXT), :], semt)
                  for bi in range(per_w)]
            for h in hs:
                h.wait()
            return carry

        lax.fori_loop(0, S // TXT, text_body, 0)

    @pl.when(wid >= half)
    def _visual():
        vw = wid - half
        pltpu.sync_copy(vp0_hbm, vp_v)
        for bi in range(per_w):
            b = vw * per_w + bi
            pltpu.sync_copy(ita0_hbm.at[b], idx0_v)
            pltpu.sync_copy(ita1_hbm.at[b], idx1_v)

            def chunk_body(c, carry, b=b):
                off = pl.multiple_of(c * CH, CH)
                g0 = pltpu.async_copy(pos_hbm.at[idx0_v.at[pl.ds(off, CH)]],
                                      r0_v, sem0)
                g1 = pltpu.async_copy(pos_hbm.at[idx1_v.at[pl.ds(off, CH)]],
                                      r1_v, sem1)
                g0.wait()
                g1.wait()

                def row_body(r, rcarry):
                    for k in range(H // LANES):
                        sl = pl.ds(k * LANES, LANES)
                        res_v[r, sl] = ((r0_v[r, sl] + r1_v[r, sl]) * 0.5
                                        + vp_v[sl])
                    return rcarry

                lax.fori_loop(0, CH, row_body, 0)
                pltpu.sync_copy(res_v, out_hbm.at[b, pl.ds(S + off, CH), :])
                return carry

            lax.fori_loop(0, V // CH, chunk_body, 0)


def _make_sc_call():
    mesh = plsc.VectorSubcoreMesh(core_axis_name="c", subcore_axis_name="s")
    return pl.kernel(
        _sc_body,
        mesh=mesh,
        out_type=jax.ShapeDtypeStruct((B, S + V, H), jnp.float32),
        scratch_types=[
            pltpu.VMEM((VPAD,), jnp.int32),
            pltpu.VMEM((VPAD,), jnp.int32),
            pltpu.VMEM((CH, H), jnp.float32),
            pltpu.VMEM((CH, H), jnp.float32),
            pltpu.VMEM((CH, H), jnp.float32),
            pltpu.VMEM((TXT, H), jnp.float32),
            pltpu.VMEM((H,), jnp.float32),
            pltpu.SemaphoreType.DMA,
            pltpu.SemaphoreType.DMA,
            pltpu.SemaphoreType.DMA,
        ],
    )


def kernel(inputs_embeds, token_type_ids, visual_embeds, visual_token_type_ids,
           image_text_alignment, pos_table, tok_type_table, vis_tok_type_table,
           vis_pos_table, vis_proj_W, vis_proj_b, ln_gamma, ln_beta):
    ita = image_text_alignment.astype(jnp.int32)
    pad = jnp.zeros((B, VPAD - V), dtype=jnp.int32)
    ita0 = jnp.concatenate([ita[:, :, 0], pad], axis=1)
    ita1 = jnp.concatenate([ita[:, :, 1], pad], axis=1)

    out_pos = _make_sc_call()(pos_table, ita0, ita1, vis_pos_table[0])

    out_emb = pl.pallas_call(
        _tc_body,
        grid=(B // NB,),
        in_specs=[
            pl.BlockSpec((1, NB, S), lambda g: (g, 0, 0)),
            pl.BlockSpec((1, NB, V), lambda g: (g, 0, 0)),
            pl.BlockSpec((TV, H), lambda g: (0, 0)),
            pl.BlockSpec((TV, H), lambda g: (0, 0)),
            pl.BlockSpec((VD, H), lambda g: (0, 0)),
            pl.BlockSpec((1, H), lambda g: (0, 0)),
            pl.BlockSpec((1, H), lambda g: (0, 0)),
            pl.BlockSpec((1, H), lambda g: (0, 0)),
            pl.BlockSpec((NB, S, H), lambda g: (g, 0, 0)),
            pl.BlockSpec((NB, V, VD), lambda g: (g, 0, 0)),
        ],
        out_specs=pl.BlockSpec((NB, S + V, H), lambda g: (g, 0, 0)),
        out_shape=jax.ShapeDtypeStruct((B, S + V, H), jnp.float32),
    )(
        token_type_ids.astype(jnp.int32).reshape(B // NB, NB, S),
        visual_token_type_ids.astype(jnp.int32).reshape(B // NB, NB, V),
        tok_type_table, vis_tok_type_table, vis_proj_W,
        vis_proj_b.reshape(1, H), ln_gamma.reshape(1, H), ln_beta.reshape(1, H),
        inputs_embeds, visual_embeds,
    )

    tails = SPAD - S - CH * (V // CH)
    f0 = ita0[:, CH * (V // CH):CH * (V // CH) + tails].reshape(B * tails, 1)
    f1 = ita1[:, CH * (V // CH):CH * (V // CH) + tails].reshape(B * tails, 1)
    out_pos = pl.pallas_call(
        _tail_body,
        grid=(1,),
        in_specs=[
            pl.BlockSpec((B * tails, 1), lambda g: (0, 0)),
            pl.BlockSpec((B * tails, 1), lambda g: (0, 0)),
            pl.BlockSpec((MAXP, H), lambda g: (0, 0)),
            pl.BlockSpec((1, H), lambda g: (0, 0)),
            pl.BlockSpec(memory_space=pltpu.MemorySpace.HBM),
        ],
        out_specs=pl.BlockSpec((B, tails, H),
                               lambda g: (0, (S + CH * (V // CH)) // tails, 0)),
        out_shape=jax.ShapeDtypeStruct((B, S + V, H), jnp.float32),
        input_output_aliases={4: 0},
    )(f0, f1, pos_table, vis_pos_table[0:1], out_pos)
    return (out_emb, out_pos)


# speedup vs baseline: 1.0533x; 1.0003x over previous
import functools

import jax
import jax.numpy as jnp
from jax import lax
from jax.experimental import pallas as pl
from jax.experimental.pallas import tpu as pltpu
from jax.experimental.pallas import tpu_sc as plsc

B, S, H = 64, 512, 768
V, VD = 196, 2048
MAXP, TV = 512, 2
EPS = 1e-12
NB = 4
CH = 32
VPAD = 200
SPAD = 712
TXT = 64
LANES = 16


def _ln(x, gamma, beta):
    mean = jnp.mean(x, axis=-1, keepdims=True)
    xc = x - mean
    var = jnp.mean(xc * xc, axis=-1, keepdims=True)
    return xc * lax.rsqrt(var + EPS) * gamma + beta


def _tc_body(tt_ids_ref, vtt_ids_ref, ttt_ref, vttt_ref, w_ref, bias_ref,
             gamma_ref, beta_ref, ie_ref, ve_ref, out_emb_ref):
    gamma = gamma_ref[0]
    beta = beta_ref[0]

    ids = tt_ids_ref[0]
    m1 = (ids == 1).astype(jnp.float32)[:, :, None]
    tte = ttt_ref[0] * (1.0 - m1) + ttt_ref[1] * m1
    out_emb_ref[:, :S, :] = _ln(ie_ref[:, :, :] + tte, gamma, beta)

    vids = vtt_ids_ref[0]
    w = w_ref[:, :].astype(jnp.bfloat16)
    for k in range(NB):
        vm1 = (vids[k] == 1).astype(jnp.float32)[:, None]
        vtt = vttt_ref[0] * (1.0 - vm1) + vttt_ref[1] * vm1
        vis = jnp.dot(ve_ref[k].astype(jnp.bfloat16), w,
                      preferred_element_type=jnp.float32) + bias_ref[0]
        out_emb_ref[k, S:, :] = _ln(vis + vtt, gamma, beta)


def _tail_body(f0_ref, f1_ref, pos_ref, vp_ref, alias_ref, out_ref):
    n = f0_ref.shape[0]
    iota = lax.broadcasted_iota(jnp.int32, (n, MAXP), 1)
    cnt = ((f0_ref[:, :] == iota).astype(jnp.float32) +
           (f1_ref[:, :] == iota).astype(jnp.float32))
    vpe = jnp.dot(cnt, pos_ref[:, :],
                  preferred_element_type=jnp.float32) * 0.5 + vp_ref[0]
    out_ref[:, :, :] = vpe.reshape(B, n // B, H)


def _sc_body(pos_hbm, ita0_hbm, ita1_hbm, vp0_hbm, out_hbm,
             idx0_v, idx1_v, r0_v, r1_v, res_v, txt_v, vp_v,
             sem0, sem1, semt):
    info = plsc.get_sparse_core_info()
    nc = info.num_cores
    nw = nc * info.num_subcores
    wid = lax.axis_index("s") * nc + lax.axis_index("c")
    half = nw // 2
    per_w = B // half


    @pl.when(wid < half)
    def _text():
        def text_body(t, carry):
            toff = pl.multiple_of(t * TXT, TXT)
            pltpu.sync_copy(pos_hbm.at[pl.ds(toff, TXT), :], txt_v)
            hs = [pltpu.async_copy(txt_v,
                                   out_hbm.at[wid * per_w + bi,
                                              pl.ds(toff, TXT), :], semt)
                  for bi in range(per_w)]
            for h in hs:
                h.wait()
            return carry

        lax.fori_loop(0, S // TXT, text_body, 0)

    @pl.when(wid >= half)
    def _visual():
        vw = wid - half
        pltpu.sync_copy(vp0_hbm, vp_v)
        for bi in range(per_w):
            b = vw * per_w + bi
            pltpu.sync_copy(ita0_hbm.at[b], idx0_v)
            pltpu.sync_copy(ita1_hbm.at[b], idx1_v)

            def chunk_body(c, carry, b=b):
                off = pl.multiple_of(c * CH, CH)
                g0 = pltpu.async_copy(pos_hbm.at[idx0_v.at[pl.ds(off, CH)]],
                                      r0_v, sem0)
                g1 = pltpu.async_copy(pos_hbm.at[idx1_v.at[pl.ds(off, CH)]],
                                      r1_v, sem1)
                g0.wait()
                g1.wait()

                def row_body(r, rcarry):
                    for k in range(H // LANES):
                        sl = pl.ds(k * LANES, LANES)
                        res_v[r, sl] = ((r0_v[r, sl] + r1_v[r, sl]) * 0.5
                                        + vp_v[sl])
                    return rcarry

                lax.fori_loop(0, CH, row_body, 0)
                pltpu.sync_copy(res_v, out_hbm.at[b, pl.ds(S + off, CH), :])
                return carry

            lax.fori_loop(0, V // CH, chunk_body, 0)


def _make_sc_call():
    mesh = plsc.VectorSubcoreMesh(core_axis_name="c", subcore_axis_name="s")
    return pl.kernel(
        _sc_body,
        mesh=mesh,
        out_type=jax.ShapeDtypeStruct((B, S + V, H), jnp.float32),
        scratch_types=[
            pltpu.VMEM((VPAD,), jnp.int32),
            pltpu.VMEM((VPAD,), jnp.int32),
            pltpu.VMEM((CH, H), jnp.float32),
            pltpu.VMEM((CH, H), jnp.float32),
            pltpu.VMEM((CH, H), jnp.float32),
            pltpu.VMEM((TXT, H), jnp.float32),
            pltpu.VMEM((H,), jnp.float32),
            pltpu.SemaphoreType.DMA,
            pltpu.SemaphoreType.DMA,
            pltpu.SemaphoreType.DMA,
        ],
    )


def kernel(inputs_embeds, token_type_ids, visual_embeds, visual_token_type_ids,
           image_text_alignment, pos_table, tok_type_table, vis_tok_type_table,
           vis_pos_table, vis_proj_W, vis_proj_b, ln_gamma, ln_beta):
    ita = image_text_alignment.astype(jnp.int32)
    pad = jnp.zeros((B, VPAD - V), dtype=jnp.int32)
    ita0 = jnp.concatenate([ita[:, :, 0], pad], axis=1)
    ita1 = jnp.concatenate([ita[:, :, 1], pad], axis=1)

    out_pos = _make_sc_call()(pos_table, ita0, ita1, vis_pos_table[0])

    out_emb = pl.pallas_call(
        _tc_body,
        grid=(B // NB,),
        in_specs=[
            pl.BlockSpec((1, NB, S), lambda g: (g, 0, 0)),
            pl.BlockSpec((1, NB, V), lambda g: (g, 0, 0)),
            pl.BlockSpec((TV, H), lambda g: (0, 0)),
            pl.BlockSpec((TV, H), lambda g: (0, 0)),
            pl.BlockSpec((VD, H), lambda g: (0, 0)),
            pl.BlockSpec((1, H), lambda g: (0, 0)),
            pl.BlockSpec((1, H), lambda g: (0, 0)),
            pl.BlockSpec((1, H), lambda g: (0, 0)),
            pl.BlockSpec((NB, S, H), lambda g: (g, 0, 0)),
            pl.BlockSpec((NB, V, VD), lambda g: (g, 0, 0)),
        ],
        out_specs=pl.BlockSpec((NB, S + V, H), lambda g: (g, 0, 0)),
        out_shape=jax.ShapeDtypeStruct((B, S + V, H), jnp.float32),
    )(
        token_type_ids.astype(jnp.int32).reshape(B // NB, NB, S),
        visual_token_type_ids.astype(jnp.int32).reshape(B // NB, NB, V),
        tok_type_table, vis_tok_type_table, vis_proj_W,
        vis_proj_b.reshape(1, H), ln_gamma.reshape(1, H), ln_beta.reshape(1, H),
        inputs_embeds, visual_embeds,
    )

    tails = SPAD - S - CH * (V // CH)
    f0 = ita0[:, CH * (V // CH):CH * (V // CH) + tails].reshape(B * tails, 1)
    f1 = ita1[:, CH * (V // CH):CH * (V // CH) + tails].reshape(B * tails, 1)
    out_pos = pl.pallas_call(
        _tail_body,
        grid=(1,),
        in_specs=[
            pl.BlockSpec((B * tails, 1), lambda g: (0, 0)),
            pl.BlockSpec((B * tails, 1), lambda g: (0, 0)),
            pl.BlockSpec((MAXP, H), lambda g: (0, 0)),
            pl.BlockSpec((1, H), lambda g: (0, 0)),
            pl.BlockSpec(memory_space=pltpu.MemorySpace.HBM),
        ],
        out_specs=pl.BlockSpec((B, tails, H),
                               lambda g: (0, (S + CH * (V // CH)) // tails, 0)),
        out_shape=jax.ShapeDtypeStruct((B, S + V, H), jnp.float32),
        input_output_aliases={4: 0},
    )(f0, f1, pos_table, vis_pos_table[0:1], out_pos)
    return (out_emb, out_pos)
